# baseline (speedup 1.0000x reference)
import functools

import jax
import jax.numpy as jnp
from jax.experimental import pallas as pl
from jax.experimental.pallas import tpu as pltpu

_R2 = 0.25
_KMAX = 128
_LN_EPS = 1e-5
_BITER = 14
_BR = 128
_CH = 512
_BLK_PREP = 512
_AUGC = 256
_XMARGIN = 0.65


def _prep_body(x_ref, p8_ref, w1_ref, w38_ref, b_ref,
               acaug_ref, acbf_ref, va_ref, bc_ref, vb_ref):
    d = x_ref.shape[1]
    p8 = p8_ref[:]
    bm = jnp.dot(p8, w38_ref[:], preferred_element_type=jnp.float32)
    a = jnp.dot(x_ref[:], w1_ref[:], preferred_element_type=jnp.float32)
    a = a + bm + b_ref[:]
    mu = jnp.mean(a, axis=1, keepdims=True)
    acv = a - mu
    acb = acv.astype(jnp.bfloat16)
    acbf_ref[:] = acb
    acaug_ref[:] = jnp.concatenate(
        [acb, jnp.ones_like(acb[:, :1]),
         jnp.zeros_like(acb[:, : _AUGC - d - 1])], axis=1)
    va_ref[:] = jnp.mean(acv * acv, axis=1, keepdims=True)
    mub = jnp.mean(bm, axis=1, keepdims=True)
    bcv = bm - mub
    bc_ref[:] = bcv
    vb_ref[:] = jnp.mean(bcv * bcv, axis=1, keepdims=True)


def _main_body(n_real, br, npad, d, nch,
               flags_ref, pb_ref, posT_ref, acaug_ref, acTbf_ref, va_ref,
               bc_ref, vb_ref, gam_ref, bet_ref, out_ref,
               d2_ref, pc_ref, s1_ref, cn_ref, hi_ref):
    i = pl.program_id(0)
    pb = pb_ref[:]
    sq_blk = jnp.sum(pb * pb, axis=1, keepdims=True)
    kf = jnp.float32(_KMAX + 1)
    pc_ref[:] = jnp.zeros_like(pc_ref)

    for c in range(nch):
        @pl.when(flags_ref[i, c] != 0)
        def _(c=c):
            pT = posT_ref[:, c * _CH:(c + 1) * _CH]
            sq_row = jnp.sum(pT * pT, axis=0, keepdims=True)
            d2c = sq_blk + sq_row - 2.0 * jnp.dot(
                pb, pT, preferred_element_type=jnp.float32)
            colc = jax.lax.broadcasted_iota(jnp.int32, (br, _CH), 1) + c * _CH
            rowc = jax.lax.broadcasted_iota(jnp.int32, (br, _CH), 0) + i * br
            d2c = jnp.where(colc == rowc, jnp.float32(-1e9), d2c)
            d2_ref[:, c * _CH:(c + 1) * _CH] = d2c
            pc_ref[:, c:c + 1] = jnp.sum(
                (d2c <= jnp.float32(_R2)).astype(jnp.float32),
                axis=1, keepdims=True)

    cnt0 = jnp.sum(pc_ref[:], axis=1, keepdims=True)
    hi_ref[:] = jnp.full((br, 1), _R2, jnp.float32)

    @pl.when(jnp.max(cnt0) >= kf)
    def _search():
        def bs_body(_, carry):
            lo, hi = carry
            mid = 0.5 * (lo + hi)
            for c in range(nch):
                @pl.when(flags_ref[i, c] != 0)
                def _(c=c):
                    pc_ref[:, c:c + 1] = jnp.sum(
                        (d2_ref[:, c * _CH:(c + 1) * _CH] <= mid).astype(
                            jnp.float32), axis=1, keepdims=True)
            cnt = jnp.sum(pc_ref[:], axis=1, keepdims=True)
            ge = cnt >= kf
            return (jnp.where(ge, lo, mid), jnp.where(ge, mid, hi))

        lo0 = jnp.full((br, 1), -1e-3, jnp.float32)
        hi0 = jnp.full((br, 1), _R2, jnp.float32)
        _, hi = jax.lax.fori_loop(0, _BITER, bs_body, (lo0, hi0))
        hi_ref[:] = hi

    hi = hi_ref[:]
    s1_ref[:] = jnp.zeros_like(s1_ref)
    cn_ref[:] = jnp.zeros_like(cn_ref)
    bcb = bc_ref[:].astype(jnp.bfloat16)
    for c in range(nch):
        @pl.when(flags_ref[i, c] != 0)
        def _(c=c):
            gch = jnp.dot(bcb, acTbf_ref[:, c * _CH:(c + 1) * _CH],
                          preferred_element_type=jnp.float32)
            maskc = d2_ref[:, c * _CH:(c + 1) * _CH] <= hi
            cn_ref[:] = cn_ref[:] + jnp.sum(
                maskc.astype(jnp.float32), axis=1, keepdims=True)
            varc = (va_ref[:, c * _CH:(c + 1) * _CH] + vb_ref[:]
                    - (2.0 / d) * gch)
            wc = jnp.where(maskc, jax.lax.rsqrt(varc + _LN_EPS),
                           0.0).astype(jnp.bfloat16)
            s1_ref[:] = s1_ref[:] + jnp.dot(
                wc, acaug_ref[c * _CH:(c + 1) * _CH, :],
                preferred_element_type=jnp.float32)

    s1 = s1_ref[:, :d]
    s0 = s1_ref[:, d:d + 1]
    cnt = cn_ref[:]
    o = (s1 - s0 * bc_ref[:]) * (gam_ref[:] / jnp.maximum(cnt, 1.0)) + bet_ref[:]
    out_ref[:] = jnp.maximum(o, 0.0)


def kernel(x, pos, batch, W, b, gamma, beta):
    n, d = x.shape
    lcm = max(_BLK_PREP, _CH, _BR)
    npad = ((n + lcm - 1) // lcm) * lcm
    nex = npad - n
    perm = jnp.argsort(pos[:, 0])
    inv = jnp.argsort(perm)
    x = x[perm]
    pos = pos[perm]
    xp = jnp.pad(x, ((0, nex), (0, 0)))
    far = 1000.0 + 100.0 * jnp.arange(nex, dtype=jnp.float32)
    p_pad = jnp.concatenate([pos, jnp.broadcast_to(far[:, None], (nex, 3))], 0)
    p8 = jnp.pad(p_pad, ((0, 0), (0, 5)))
    posT = p8.T
    w1 = W[:d]
    w38 = jnp.pad(W[d:], ((0, 5), (0, 0)))
    b_row = b.reshape(1, d)
    gam = gamma.reshape(1, d)
    bet = beta.reshape(1, d)

    xsp = p_pad[:, 0]
    bmin = xsp[::_BR]
    bmax = xsp[_BR - 1::_BR]
    cmin = xsp[::_CH]
    cmax = xsp[_CH - 1::_CH]
    lim = jnp.float32(0.5 + _XMARGIN)
    flags = ((cmin[None, :] <= bmax[:, None] + lim)
             & (cmax[None, :] >= bmin[:, None] - lim)).astype(jnp.int32)

    acaug, acbf, va, bc, vb = pl.pallas_call(
        _prep_body,
        grid=(npad // _BLK_PREP,),
        in_specs=[
            pl.BlockSpec((_BLK_PREP, d), lambda i: (i, 0)),
            pl.BlockSpec((_BLK_PREP, 8), lambda i: (i, 0)),
            pl.BlockSpec((d, d), lambda i: (0, 0)),
            pl.BlockSpec((8, d), lambda i: (0, 0)),
            pl.BlockSpec((1, d), lambda i: (0, 0)),
        ],
        out_specs=[
            pl.BlockSpec((_BLK_PREP, _AUGC), lambda i: (i, 0)),
            pl.BlockSpec((_BLK_PREP, d), lambda i: (i, 0)),
            pl.BlockSpec((_BLK_PREP, 1), lambda i: (i, 0)),
            pl.BlockSpec((_BLK_PREP, d), lambda i: (i, 0)),
            pl.BlockSpec((_BLK_PREP, 1), lambda i: (i, 0)),
        ],
        out_shape=[
            jax.ShapeDtypeStruct((npad, _AUGC), jnp.bfloat16),
            jax.ShapeDtypeStruct((npad, d), jnp.bfloat16),
            jax.ShapeDtypeStruct((npad, 1), jnp.float32),
            jax.ShapeDtypeStruct((npad, d), jnp.float32),
            jax.ShapeDtypeStruct((npad, 1), jnp.float32),
        ],
    )(xp, p8, w1, w38, b_row)

    acTbf = acbf.T
    va_row = va.reshape(1, npad)
    nch = npad // _CH

    out = pl.pallas_call(
        functools.partial(_main_body, n, _BR, npad, d, nch),
        grid_spec=pltpu.PrefetchScalarGridSpec(
            num_scalar_prefetch=1,
            grid=(npad // _BR,),
            in_specs=[
                pl.BlockSpec((_BR, 8), lambda i, f: (i, 0)),
                pl.BlockSpec((8, npad), lambda i, f: (0, 0)),
                pl.BlockSpec((npad, _AUGC), lambda i, f: (0, 0)),
                pl.BlockSpec((d, npad), lambda i, f: (0, 0)),
                pl.BlockSpec((1, npad), lambda i, f: (0, 0)),
                pl.BlockSpec((_BR, d), lambda i, f: (i, 0)),
                pl.BlockSpec((_BR, 1), lambda i, f: (i, 0)),
                pl.BlockSpec((1, d), lambda i, f: (0, 0)),
                pl.BlockSpec((1, d), lambda i, f: (0, 0)),
            ],
            out_specs=pl.BlockSpec((_BR, d), lambda i, f: (i, 0)),
            scratch_shapes=[
                pltpu.VMEM((_BR, npad), jnp.float32),
                pltpu.VMEM((_BR, 128), jnp.float32),
                pltpu.VMEM((_BR, _AUGC), jnp.float32),
                pltpu.VMEM((_BR, 1), jnp.float32),
                pltpu.VMEM((_BR, 1), jnp.float32),
            ],
        ),
        out_shape=jax.ShapeDtypeStruct((npad, d), jnp.float32),
    )(flags, p8, posT, acaug, acTbf, va_row, bc, vb, gam, bet)

    return out[:n][inv]


# speedup vs baseline: 1.6744x; 1.6744x over previous
import functools

import jax
import jax.numpy as jnp
from jax.experimental import pallas as pl
from jax.experimental.pallas import tpu as pltpu

_R2 = 0.25
_KMAX = 128
_LN_EPS = 1e-5
_BITER = 14
_BR = 128
_CH = 2048
_BLK_PREP = 512
_AUGC = 256
_XMARGIN = 0.65


def _prep_body(x_ref, p8_ref, w1_ref, w38_ref, b_ref,
               acaug_ref, acbf_ref, va_ref, bc_ref, vb_ref):
    d = x_ref.shape[1]
    p8 = p8_ref[:]
    bm = jnp.dot(p8, w38_ref[:], preferred_element_type=jnp.float32)
    a = jnp.dot(x_ref[:], w1_ref[:], preferred_element_type=jnp.float32)
    a = a + bm + b_ref[:]
    mu = jnp.mean(a, axis=1, keepdims=True)
    acv = a - mu
    acb = acv.astype(jnp.bfloat16)
    acbf_ref[:] = acb
    acaug_ref[:] = jnp.concatenate(
        [acb, jnp.ones_like(acb[:, :1]),
         jnp.zeros_like(acb[:, : _AUGC - d - 1])], axis=1)
    va_ref[:] = jnp.mean(acv * acv, axis=1, keepdims=True)
    mub = jnp.mean(bm, axis=1, keepdims=True)
    bcv = bm - mub
    bc_ref[:] = bcv
    vb_ref[:] = jnp.mean(bcv * bcv, axis=1, keepdims=True)


def _main_body(n_real, br, npad, d, nch,
               flags_ref, pb_ref, posT_ref, acaug_ref, acTbf_ref, va_ref,
               bc_ref, vb_ref, gam_ref, bet_ref, out_ref,
               d2_ref, pc_ref, s1_ref, cn_ref, hi_ref):
    i = pl.program_id(0)
    pb = pb_ref[:]
    sq_blk = jnp.sum(pb * pb, axis=1, keepdims=True)
    kf = jnp.float32(_KMAX + 1)
    pc_ref[:] = jnp.zeros_like(pc_ref)

    for c in range(nch):
        @pl.when(flags_ref[i, c] != 0)
        def _(c=c):
            pT = posT_ref[:, c * _CH:(c + 1) * _CH]
            sq_row = jnp.sum(pT * pT, axis=0, keepdims=True)
            d2c = sq_blk + sq_row - 2.0 * jnp.dot(
                pb, pT, preferred_element_type=jnp.float32)
            colc = jax.lax.broadcasted_iota(jnp.int32, (br, _CH), 1) + c * _CH
            rowc = jax.lax.broadcasted_iota(jnp.int32, (br, _CH), 0) + i * br
            d2c = jnp.where(colc == rowc, jnp.float32(-1e9), d2c)
            d2_ref[:, c * _CH:(c + 1) * _CH] = d2c
            pc_ref[:, c:c + 1] = jnp.sum(
                (d2c <= jnp.float32(_R2)).astype(jnp.float32),
                axis=1, keepdims=True)

    cnt0 = jnp.sum(pc_ref[:], axis=1, keepdims=True)
    hi_ref[:] = jnp.full((br, 1), _R2, jnp.float32)

    @pl.when(jnp.max(cnt0) >= kf)
    def _search():
        def bs_body(_, carry):
            lo, hi = carry
            mid = 0.5 * (lo + hi)
            for c in range(nch):
                @pl.when(flags_ref[i, c] != 0)
                def _(c=c):
                    pc_ref[:, c:c + 1] = jnp.sum(
                        (d2_ref[:, c * _CH:(c + 1) * _CH] <= mid).astype(
                            jnp.float32), axis=1, keepdims=True)
            cnt = jnp.sum(pc_ref[:], axis=1, keepdims=True)
            ge = cnt >= kf
            return (jnp.where(ge, lo, mid), jnp.where(ge, mid, hi))

        lo0 = jnp.full((br, 1), -1e-3, jnp.float32)
        hi0 = jnp.full((br, 1), _R2, jnp.float32)
        _, hi = jax.lax.fori_loop(0, _BITER, bs_body, (lo0, hi0))
        hi_ref[:] = hi

    hi = hi_ref[:]
    s1_ref[:] = jnp.zeros_like(s1_ref)
    cn_ref[:] = jnp.zeros_like(cn_ref)
    bcb = bc_ref[:].astype(jnp.bfloat16)
    for c in range(nch):
        @pl.when(flags_ref[i, c] != 0)
        def _(c=c):
            gch = jnp.dot(bcb, acTbf_ref[:, c * _CH:(c + 1) * _CH],
                          preferred_element_type=jnp.float32)
            maskc = d2_ref[:, c * _CH:(c + 1) * _CH] <= hi
            cn_ref[:] = cn_ref[:] + jnp.sum(
                maskc.astype(jnp.float32), axis=1, keepdims=True)
            varc = (va_ref[:, c * _CH:(c + 1) * _CH] + vb_ref[:]
                    - (2.0 / d) * gch)
            wc = jnp.where(maskc, jax.lax.rsqrt(varc + _LN_EPS),
                           0.0).astype(jnp.bfloat16)
            s1_ref[:] = s1_ref[:] + jnp.dot(
                wc, acaug_ref[c * _CH:(c + 1) * _CH, :],
                preferred_element_type=jnp.float32)

    s1 = s1_ref[:, :d]
    s0 = s1_ref[:, d:d + 1]
    cnt = cn_ref[:]
    o = (s1 - s0 * bc_ref[:]) * (gam_ref[:] / jnp.maximum(cnt, 1.0)) + bet_ref[:]
    out_ref[:] = jnp.maximum(o, 0.0)


def kernel(x, pos, batch, W, b, gamma, beta):
    n, d = x.shape
    lcm = max(_BLK_PREP, _CH, _BR)
    npad = ((n + lcm - 1) // lcm) * lcm
    nex = npad - n
    perm = jnp.argsort(pos[:, 0])
    inv = jnp.argsort(perm)
    x = x[perm]
    pos = pos[perm]
    xp = jnp.pad(x, ((0, nex), (0, 0)))
    far = 1000.0 + 100.0 * jnp.arange(nex, dtype=jnp.float32)
    p_pad = jnp.concatenate([pos, jnp.broadcast_to(far[:, None], (nex, 3))], 0)
    p8 = jnp.pad(p_pad, ((0, 0), (0, 5)))
    posT = p8.T
    w1 = W[:d]
    w38 = jnp.pad(W[d:], ((0, 5), (0, 0)))
    b_row = b.reshape(1, d)
    gam = gamma.reshape(1, d)
    bet = beta.reshape(1, d)

    xsp = p_pad[:, 0]
    bmin = xsp[::_BR]
    bmax = xsp[_BR - 1::_BR]
    cmin = xsp[::_CH]
    cmax = xsp[_CH - 1::_CH]
    lim = jnp.float32(0.5 + _XMARGIN)
    flags = ((cmin[None, :] <= bmax[:, None] + lim)
             & (cmax[None, :] >= bmin[:, None] - lim)).astype(jnp.int32)

    acaug, acbf, va, bc, vb = pl.pallas_call(
        _prep_body,
        grid=(npad // _BLK_PREP,),
        in_specs=[
            pl.BlockSpec((_BLK_PREP, d), lambda i: (i, 0)),
            pl.BlockSpec((_BLK_PREP, 8), lambda i: (i, 0)),
            pl.BlockSpec((d, d), lambda i: (0, 0)),
            pl.BlockSpec((8, d), lambda i: (0, 0)),
            pl.BlockSpec((1, d), lambda i: (0, 0)),
        ],
        out_specs=[
            pl.BlockSpec((_BLK_PREP, _AUGC), lambda i: (i, 0)),
            pl.BlockSpec((_BLK_PREP, d), lambda i: (i, 0)),
            pl.BlockSpec((_BLK_PREP, 1), lambda i: (i, 0)),
            pl.BlockSpec((_BLK_PREP, d), lambda i: (i, 0)),
            pl.BlockSpec((_BLK_PREP, 1), lambda i: (i, 0)),
        ],
        out_shape=[
            jax.ShapeDtypeStruct((npad, _AUGC), jnp.bfloat16),
            jax.ShapeDtypeStruct((npad, d), jnp.bfloat16),
            jax.ShapeDtypeStruct((npad, 1), jnp.float32),
            jax.ShapeDtypeStruct((npad, d), jnp.float32),
            jax.ShapeDtypeStruct((npad, 1), jnp.float32),
        ],
    )(xp, p8, w1, w38, b_row)

    acTbf = acbf.T
    va_row = va.reshape(1, npad)
    nch = npad // _CH

    out = pl.pallas_call(
        functools.partial(_main_body, n, _BR, npad, d, nch),
        grid_spec=pltpu.PrefetchScalarGridSpec(
            num_scalar_prefetch=1,
            grid=(npad // _BR,),
            in_specs=[
                pl.BlockSpec((_BR, 8), lambda i, f: (i, 0)),
                pl.BlockSpec((8, npad), lambda i, f: (0, 0)),
                pl.BlockSpec((npad, _AUGC), lambda i, f: (0, 0)),
                pl.BlockSpec((d, npad), lambda i, f: (0, 0)),
                pl.BlockSpec((1, npad), lambda i, f: (0, 0)),
                pl.BlockSpec((_BR, d), lambda i, f: (i, 0)),
                pl.BlockSpec((_BR, 1), lambda i, f: (i, 0)),
                pl.BlockSpec((1, d), lambda i, f: (0, 0)),
                pl.BlockSpec((1, d), lambda i, f: (0, 0)),
            ],
            out_specs=pl.BlockSpec((_BR, d), lambda i, f: (i, 0)),
            scratch_shapes=[
                pltpu.VMEM((_BR, npad), jnp.float32),
                pltpu.VMEM((_BR, 128), jnp.float32),
                pltpu.VMEM((_BR, _AUGC), jnp.float32),
                pltpu.VMEM((_BR, 1), jnp.float32),
                pltpu.VMEM((_BR, 1), jnp.float32),
            ],
        ),
        out_shape=jax.ShapeDtypeStruct((npad, d), jnp.float32),
    )(flags, p8, posT, acaug, acTbf, va_row, bc, vb, gam, bet)

    return out[:n][inv]
